# baseline (device time: 33615 ns/iter reference)
import functools

import jax
import jax.numpy as jnp
from jax import lax
from jax.experimental import pallas as pl
from jax.experimental.pallas import tpu as pltpu

N_DEV = 8
TILE = 8
N_TILES = 2048 // TILE


def kernel(x):
    m, n = x.shape
    x3 = x.reshape(N_TILES, TILE, n)

    def body(x_ref, out_ref, send_row, totals_buf, send_sems, recv_sems):
        my_pos = lax.axis_index("i")

        barrier_sem = pltpu.get_barrier_semaphore()
        for off in range(1, N_DEV):
            pl.semaphore_signal(
                barrier_sem, inc=1,
                device_id=(lax.rem(my_pos + off, N_DEV),),
                device_id_type=pl.DeviceIdType.MESH,
            )
        pl.semaphore_wait(barrier_sem, N_DEV - 1)

        v = x_ref[...]
        for d in (1, 2, 4):
            shifted = jnp.concatenate(
                [
                    jnp.ones((N_TILES, d, n), jnp.float32),
                    v[:, : TILE - d, :],
                ],
                axis=1,
            )
            v = v * shifted

        inc = jnp.reshape(v[:, TILE - 1 : TILE, :], (N_TILES, n))
        s = 1
        while s < N_TILES:
            shifted = jnp.concatenate(
                [jnp.ones((s, n), jnp.float32), inc[: N_TILES - s]], axis=0
            )
            inc = inc * shifted
            s *= 2
        gp = jnp.reshape(
            jnp.concatenate(
                [jnp.ones((1, n), jnp.float32), inc[: N_TILES - 1]], axis=0
            ),
            (N_TILES, 1, n),
        )
        send_row[...] = jnp.reshape(inc[N_TILES - 1 : N_TILES], (1, 1, n))

        descs = []
        for o in range(1, N_DEV):
            rdma = pltpu.make_async_remote_copy(
                src_ref=send_row,
                dst_ref=totals_buf.at[pl.ds(o, 1)],
                send_sem=send_sems.at[o],
                recv_sem=recv_sems.at[o],
                device_id=(lax.rem(my_pos + o, N_DEV),),
                device_id_type=pl.DeviceIdType.MESH,
            )
            descs.append(rdma)

            @pl.when(my_pos + o < N_DEV)
            def _():
                rdma.start()

        for o in range(1, N_DEV):
            rdma = descs[o - 1]

            @pl.when(o <= my_pos)
            def _():
                rdma.wait_recv()

        row = lax.broadcasted_iota(jnp.int32, (N_DEV, 1, n), 0)
        mask = (row >= 1) & (row <= my_pos)
        t = jnp.where(
            mask, totals_buf[...], jnp.ones((N_DEV, 1, n), jnp.float32)
        )
        t = t[0:4] * t[4:8]
        t = t[0:2] * t[2:4]
        pre = t[0:1] * t[1:2]

        out_ref[...] = v * (gp * pre)

        for o in range(1, N_DEV):
            rdma = descs[o - 1]

            @pl.when(my_pos + o < N_DEV)
            def _():
                rdma.wait_send()

        @functools.partial(
            pl.run_scoped, second_barrier=pltpu.SemaphoreType.REGULAR
        )
        def _(second_barrier):
            for off in range(1, N_DEV):
                pl.semaphore_signal(
                    second_barrier, inc=1,
                    device_id=(lax.rem(my_pos + off, N_DEV),),
                    device_id_type=pl.DeviceIdType.MESH,
                )
            pl.semaphore_wait(second_barrier, N_DEV - 1)

    out3 = pl.pallas_call(
        body,
        out_shape=jax.ShapeDtypeStruct((N_TILES, TILE, n), jnp.float32),
        in_specs=[pl.BlockSpec(memory_space=pltpu.VMEM)],
        out_specs=pl.BlockSpec(memory_space=pltpu.VMEM),
        scratch_shapes=[
            pltpu.VMEM((1, 1, n), jnp.float32),
            pltpu.VMEM((N_DEV, 1, n), jnp.float32),
            pltpu.SemaphoreType.DMA((N_DEV,)),
            pltpu.SemaphoreType.DMA((N_DEV,)),
        ],
        compiler_params=pltpu.CompilerParams(collective_id=0),
    )(x3)
    return out3.reshape(m, n)


# device time: 19625 ns/iter; 1.7129x vs baseline; 1.7129x over previous
import functools

import jax
import jax.numpy as jnp
from jax import lax
from jax.experimental import pallas as pl
from jax.experimental.pallas import tpu as pltpu

N_DEV = 8
GROUP = 32
N_GROUPS = 2048 // GROUP
PRE_GROUPS = 26


def kernel(x):
    m, n = x.shape

    def body(x_ref, out_ref, send_row, totals_buf, send_sems, recv_sems):
        my_pos = lax.axis_index("i")

        barrier_sem = pltpu.get_barrier_semaphore()
        for off in range(1, N_DEV):
            pl.semaphore_signal(
                barrier_sem, inc=1,
                device_id=(lax.rem(my_pos + off, N_DEV),),
                device_id_type=pl.DeviceIdType.MESH,
            )
        pl.semaphore_wait(barrier_sem, N_DEV - 1)

        ones_row = jnp.ones((1, n), jnp.float32)

        gts = []
        for g in range(N_GROUPS):
            u = x_ref[pl.ds(g * GROUP, GROUP), :]
            r = GROUP
            while r > 1:
                u = u[: r // 2] * u[r // 2 : r]
                r //= 2
            gts.append(u)
        inc = jnp.concatenate(gts, axis=0)
        s = 1
        while s < N_GROUPS:
            shifted = jnp.concatenate(
                [jnp.ones((s, n), jnp.float32), inc[: N_GROUPS - s]], axis=0
            )
            inc = inc * shifted
            s *= 2
        gps = [ones_row] + [inc[g : g + 1] for g in range(N_GROUPS - 1)]
        send_row[...] = inc[N_GROUPS - 1 : N_GROUPS]

        descs = []
        for o in range(1, N_DEV):
            rdma = pltpu.make_async_remote_copy(
                src_ref=send_row,
                dst_ref=totals_buf.at[pl.ds(o, 1)],
                send_sem=send_sems.at[o],
                recv_sem=recv_sems.at[o],
                device_id=(lax.rem(my_pos + o, N_DEV),),
                device_id_type=pl.DeviceIdType.MESH,
            )
            descs.append(rdma)

            @pl.when(my_pos + o < N_DEV)
            def _():
                rdma.start()

        def scan_group(g, carry):
            v = x_ref[pl.ds(g * GROUP, GROUP), :]
            d = 1
            while d < GROUP:
                shifted = jnp.concatenate(
                    [jnp.ones((d, n), jnp.float32), v[: GROUP - d]], axis=0
                )
                v = v * shifted
                d *= 2
            out_ref[pl.ds(g * GROUP, GROUP), :] = v * carry

        for g in range(PRE_GROUPS):
            scan_group(g, gps[g])

        for o in range(1, N_DEV):
            rdma = descs[o - 1]

            @pl.when(o <= my_pos)
            def _():
                rdma.wait_recv()

        row = lax.broadcasted_iota(jnp.int32, (N_DEV, n), 0)
        mask = (row >= 1) & (row <= my_pos)
        t = jnp.where(mask, totals_buf[...], jnp.ones((N_DEV, n), jnp.float32))
        t = t[0:4] * t[4:8]
        t = t[0:2] * t[2:4]
        pre = t[0:1] * t[1:2]

        for g in range(PRE_GROUPS, N_GROUPS):
            scan_group(g, gps[g] * pre)

        for g in range(PRE_GROUPS):
            out_ref[pl.ds(g * GROUP, GROUP), :] = (
                out_ref[pl.ds(g * GROUP, GROUP), :] * pre
            )

        for o in range(1, N_DEV):
            rdma = descs[o - 1]

            @pl.when(my_pos + o < N_DEV)
            def _():
                rdma.wait_send()

        @functools.partial(
            pl.run_scoped, second_barrier=pltpu.SemaphoreType.REGULAR
        )
        def _(second_barrier):
            for off in range(1, N_DEV):
                pl.semaphore_signal(
                    second_barrier, inc=1,
                    device_id=(lax.rem(my_pos + off, N_DEV),),
                    device_id_type=pl.DeviceIdType.MESH,
                )
            pl.semaphore_wait(second_barrier, N_DEV - 1)

    return pl.pallas_call(
        body,
        out_shape=jax.ShapeDtypeStruct((m, n), jnp.float32),
        in_specs=[pl.BlockSpec(memory_space=pltpu.VMEM)],
        out_specs=pl.BlockSpec(memory_space=pltpu.VMEM),
        scratch_shapes=[
            pltpu.VMEM((1, n), jnp.float32),
            pltpu.VMEM((N_DEV, n), jnp.float32),
            pltpu.SemaphoreType.DMA((N_DEV,)),
            pltpu.SemaphoreType.DMA((N_DEV,)),
        ],
        compiler_params=pltpu.CompilerParams(collective_id=0),
    )(x)
